# baseline (speedup 1.0000x reference)
import jax
import jax.numpy as jnp
from jax.experimental import pallas as pl

M = 2048
WROW = 256
NE = 864
RPAD = 248


def _fmt_body(x_ref, o_ref):
    k = pl.program_id(0)

    @pl.when((k >= 1) & (k <= 28))
    def _():
        v = x_ref[...]
        z1 = jnp.zeros((96, 8, 1), jnp.float32)
        z31 = jnp.zeros((96, 8, 31), jnp.float32)
        vp = jnp.concatenate([z1, v, z31], axis=2)
        hi = vp.astype(jnp.bfloat16)
        lo = (vp - hi.astype(jnp.float32)).astype(jnp.bfloat16)
        o_ref[...] = jnp.concatenate([hi, lo], 0)

    @pl.when((k < 1) | (k > 28))
    def _():
        o_ref[...] = jnp.zeros((192, 8, WROW), jnp.bfloat16)


def _moe_body(x_ref, w_ref, ebt_ref, sb_ref, o_ref):
    i = pl.program_id(0)
    base = pl.multiple_of(i * M + 1664, 128)
    xw = x_ref[:, pl.ds(base, M + 768)]
    var = [jax.lax.slice(xw, (0, 127 + dx), (192, 127 + dx + M + 512))
           for dx in range(3)]
    his, los = [], []
    for dy in range(3):
        for dx in range(3):
            v = var[dx]
            his.append(jax.lax.slice(v, (0, dy * 256), (96, dy * 256 + M)))
            los.append(jax.lax.slice(v, (96, dy * 256), (192, dy * 256 + M)))
    x9 = jnp.concatenate(his, 0)
    x9l = jnp.concatenate(los, 0)
    acc = jax.lax.dot_general(w_ref[...], x9, (((1,), (0,)), ((), ())),
                              preferred_element_type=jnp.float32)
    zl = jax.lax.dot_general(w_ref[NE:NE + 16], x9l, (((1,), (0,)), ((), ())),
                             preferred_element_type=jnp.float32)
    z = acc[NE:NE + 8] + acc[NE + 8:NE + 16] + zl[0:8] + zl[8:16]
    neg = jnp.float32(-1e30)
    m1 = jnp.full((1, M), neg, jnp.float32)
    i1 = jnp.zeros((1, M), jnp.int32)
    for e in range(8):
        ze = z[e:e + 1]
        c = ze > m1
        m1 = jnp.where(c, ze, m1)
        i1 = jnp.where(c, e, i1)
    m2 = jnp.full((1, M), neg, jnp.float32)
    i2 = jnp.zeros((1, M), jnp.int32)
    for e in range(8):
        ze = z[e:e + 1]
        c = (ze > m2) & (i1 != e)
        m2 = jnp.where(c, ze, m2)
        i2 = jnp.where(c, e, i2)
    s1 = 1.0 / (1.0 + jnp.exp(-m1))
    s2 = 1.0 / (1.0 + jnp.exp(-m2))
    w1 = 1.0 / (1.0 + jnp.exp(s2 - s1))
    w2 = 1.0 - w1
    eio = jax.lax.broadcasted_iota(jnp.int32, (8, M), 0)
    sv = (jnp.where(eio == i1, w1, jnp.float32(0.0))
          + jnp.where(eio == i2, w2, jnp.float32(0.0)))
    out = acc[768:864]
    for e in range(8):
        out = out + acc[e * 96:(e + 1) * 96] * sv[e:e + 1]
    out = out + jax.lax.dot_general(ebt_ref[...], sv, (((1,), (0,)), ((), ())),
                                    preferred_element_type=jnp.float32)
    out = out + sb_ref[...]
    o_ref[...] = out


def kernel(x, gate_W, expert_W, expert_b, shared_W, shared_b):
    B, Cin, H, W = x.shape
    E, Cout = expert_W.shape[0], expert_W.shape[1]
    xp = jnp.pad(x[0], ((0, 0), (8, 16), (1, 31)))
    hi = xp.astype(jnp.bfloat16)
    lo = (xp - hi.astype(jnp.float32)).astype(jnp.bfloat16)
    xbig3 = jnp.concatenate([hi, lo], 0)
    xbig = xbig3.reshape(2 * Cin, RPAD * WROW)
    ew = expert_W.reshape(E * Cout, Cin, 3, 3)
    allw = jnp.concatenate([ew, shared_W], 0)
    wflat = jnp.transpose(allw, (0, 2, 3, 1)).reshape(NE, 9 * Cin)
    g = jnp.transpose(gate_W, (0, 2, 3, 1)).reshape(E, 9 * Cin)
    g_hi = g.astype(jnp.bfloat16)
    g_lo = (g - g_hi.astype(jnp.float32)).astype(jnp.bfloat16)
    wall = jnp.concatenate(
        [wflat.astype(jnp.bfloat16), g_hi, g_lo], 0)
    ebt = expert_b.T
    sb2 = shared_b[:, None]
    nt = 28
    out_flat = pl.pallas_call(
        _moe_body,
        grid=(nt,),
        in_specs=[
            pl.BlockSpec((2 * Cin, RPAD * WROW), lambda i: (0, 0)),
            pl.BlockSpec((NE + 16, 9 * Cin), lambda i: (0, 0)),
            pl.BlockSpec((Cout, E), lambda i: (0, 0)),
            pl.BlockSpec((Cout, 1), lambda i: (0, 0)),
        ],
        out_specs=pl.BlockSpec((Cout, M), lambda i: (0, i + 1)),
        out_shape=jax.ShapeDtypeStruct((Cout, (nt + 1) * M), jnp.float32),
    )(xbig, wall, ebt, sb2)
    out = out_flat.reshape(Cout, (nt + 1) * 8, WROW)[:, 8:8 + H, 1:1 + W]
    return out[None]


# speedup vs baseline: 1.0043x; 1.0043x over previous
import jax
import jax.numpy as jnp
from jax.experimental import pallas as pl

M = 2048
WROW = 256
NE = 864
RPAD = 256


def _moe_body(x_ref, w_ref, ebt_ref, sb_ref, o_ref):
    i = pl.program_id(0)
    base = pl.multiple_of(i * M + 3712, 128)
    xw = x_ref[:, pl.ds(base, M + 768)]
    var = [jax.lax.slice(xw, (0, 127 + dx), (192, 127 + dx + M + 512))
           for dx in range(3)]
    his, los = [], []
    for dy in range(3):
        for dx in range(3):
            v = var[dx]
            his.append(jax.lax.slice(v, (0, dy * 256), (96, dy * 256 + M)))
            los.append(jax.lax.slice(v, (96, dy * 256), (192, dy * 256 + M)))
    x9 = jnp.concatenate(his, 0)
    x9l = jnp.concatenate(los, 0)
    acc = jax.lax.dot_general(w_ref[...], x9, (((1,), (0,)), ((), ())),
                              preferred_element_type=jnp.float32)
    zl = jax.lax.dot_general(w_ref[NE:NE + 16], x9l, (((1,), (0,)), ((), ())),
                             preferred_element_type=jnp.float32)
    z = acc[NE:NE + 8] + acc[NE + 8:NE + 16] + zl[0:8] + zl[8:16]
    neg = jnp.float32(-1e30)
    m1 = jnp.full((1, M), neg, jnp.float32)
    i1 = jnp.zeros((1, M), jnp.int32)
    for e in range(8):
        ze = z[e:e + 1]
        c = ze > m1
        m1 = jnp.where(c, ze, m1)
        i1 = jnp.where(c, e, i1)
    m2 = jnp.full((1, M), neg, jnp.float32)
    i2 = jnp.zeros((1, M), jnp.int32)
    for e in range(8):
        ze = z[e:e + 1]
        c = (ze > m2) & (i1 != e)
        m2 = jnp.where(c, ze, m2)
        i2 = jnp.where(c, e, i2)
    s1 = 1.0 / (1.0 + jnp.exp(-m1))
    s2 = 1.0 / (1.0 + jnp.exp(-m2))
    w1 = 1.0 / (1.0 + jnp.exp(s2 - s1))
    w2 = 1.0 - w1
    eio = jax.lax.broadcasted_iota(jnp.int32, (8, M), 0)
    sv = (jnp.where(eio == i1, w1, jnp.float32(0.0))
          + jnp.where(eio == i2, w2, jnp.float32(0.0)))
    out = acc[768:864]
    for e in range(8):
        out = out + acc[e * 96:(e + 1) * 96] * sv[e:e + 1]
    out = out + jax.lax.dot_general(ebt_ref[...], sv, (((1,), (0,)), ((), ())),
                                    precision=jax.lax.Precision.HIGHEST,
                                    preferred_element_type=jnp.float32)
    out = out + sb_ref[...]
    o_ref[...] = out


def kernel(x, gate_W, expert_W, expert_b, shared_W, shared_b):
    B, Cin, H, W = x.shape
    E, Cout = expert_W.shape[0], expert_W.shape[1]
    xpw = jnp.pad(x[0], ((0, 0), (16, 16), (1, WROW - W - 1)))
    hi = xpw.astype(jnp.bfloat16)
    lo = (xpw - hi.astype(jnp.float32)).astype(jnp.bfloat16)
    xflat = jnp.concatenate([hi, lo], 0).reshape(2 * Cin, RPAD * WROW)
    ew = expert_W.reshape(E * Cout, Cin, 3, 3)
    allw = jnp.concatenate([ew, shared_W], 0)
    wflat = jnp.transpose(allw, (0, 2, 3, 1)).reshape(NE, 9 * Cin)
    g = jnp.transpose(gate_W, (0, 2, 3, 1)).reshape(E, 9 * Cin)
    g_hi = g.astype(jnp.bfloat16)
    g_lo = (g - g_hi.astype(jnp.float32)).astype(jnp.bfloat16)
    wall = jnp.concatenate(
        [wflat.astype(jnp.bfloat16), g_hi, g_lo], 0)
    ebt = expert_b.T
    sb2 = shared_b[:, None]
    nt = 28
    out_flat = pl.pallas_call(
        _moe_body,
        grid=(nt,),
        in_specs=[
            pl.BlockSpec((2 * Cin, RPAD * WROW), lambda i: (0, 0)),
            pl.BlockSpec((NE + 16, 9 * Cin), lambda i: (0, 0)),
            pl.BlockSpec((Cout, E), lambda i: (0, 0)),
            pl.BlockSpec((Cout, 1), lambda i: (0, 0)),
        ],
        out_specs=pl.BlockSpec((Cout, M), lambda i: (0, i + 2)),
        out_shape=jax.ShapeDtypeStruct((Cout, (nt + 2) * M), jnp.float32),
    )(xflat, wall, ebt, sb2)
    out = out_flat.reshape(Cout, (nt + 2) * 8, WROW)[:, 16:16 + H, 1:1 + W]
    return out[None]


# speedup vs baseline: 1.0598x; 1.0553x over previous
import jax
import jax.numpy as jnp
from jax.experimental import pallas as pl

M = 2048
WROW = 256
NE = 864
RPAD = 256


def _moe_body(xh_ref, xl_ref, w_ref, ebt_ref, sb_ref, o_ref):
    i = pl.program_id(0)
    base = pl.multiple_of(i * M + 3712, 128)
    xwh = xh_ref[:, pl.ds(base, M + 768)]
    xwl = xl_ref[:, pl.ds(base, M + 768)]
    varh = [jax.lax.slice(xwh, (0, 127 + dx), (96, 127 + dx + M + 512))
            for dx in range(3)]
    varl = [jax.lax.slice(xwl, (0, 127 + dx), (96, 127 + dx + M + 512))
            for dx in range(3)]
    his, los = [], []
    for dy in range(3):
        for dx in range(3):
            his.append(jax.lax.slice(varh[dx], (0, dy * 256),
                                     (96, dy * 256 + M)))
            los.append(jax.lax.slice(varl[dx], (0, dy * 256),
                                     (96, dy * 256 + M)))
    x9 = jnp.concatenate(his, 0)
    x9l = jnp.concatenate(los, 0)
    acc = jax.lax.dot_general(w_ref[...], x9, (((1,), (0,)), ((), ())),
                              preferred_element_type=jnp.float32)
    zl = jax.lax.dot_general(w_ref[NE:NE + 16], x9l, (((1,), (0,)), ((), ())),
                             preferred_element_type=jnp.float32)
    z = acc[NE:NE + 8] + acc[NE + 8:NE + 16] + zl[0:8] + zl[8:16]
    neg = jnp.float32(-1e30)
    m1 = jnp.full((1, M), neg, jnp.float32)
    i1 = jnp.zeros((1, M), jnp.int32)
    for e in range(8):
        ze = z[e:e + 1]
        c = ze > m1
        m1 = jnp.where(c, ze, m1)
        i1 = jnp.where(c, e, i1)
    m2 = jnp.full((1, M), neg, jnp.float32)
    i2 = jnp.zeros((1, M), jnp.int32)
    for e in range(8):
        ze = z[e:e + 1]
        c = (ze > m2) & (i1 != e)
        m2 = jnp.where(c, ze, m2)
        i2 = jnp.where(c, e, i2)
    s1 = 1.0 / (1.0 + jnp.exp(-m1))
    s2 = 1.0 / (1.0 + jnp.exp(-m2))
    w1 = 1.0 / (1.0 + jnp.exp(s2 - s1))
    w2 = 1.0 - w1
    eio = jax.lax.broadcasted_iota(jnp.int32, (8, M), 0)
    sv = (jnp.where(eio == i1, w1, jnp.float32(0.0))
          + jnp.where(eio == i2, w2, jnp.float32(0.0)))
    out = acc[768:864]
    for e in range(8):
        out = out + acc[e * 96:(e + 1) * 96] * sv[e:e + 1]
    out = out + jax.lax.dot_general(ebt_ref[...], sv, (((1,), (0,)), ((), ())),
                                    precision=jax.lax.Precision.HIGHEST,
                                    preferred_element_type=jnp.float32)
    out = out + sb_ref[...]
    o_ref[...] = out


def kernel(x, gate_W, expert_W, expert_b, shared_W, shared_b):
    B, Cin, H, W = x.shape
    E, Cout = expert_W.shape[0], expert_W.shape[1]
    xpw = jnp.pad(x[0], ((0, 0), (16, 16), (1, WROW - W - 1)))
    hi = xpw.astype(jnp.bfloat16).reshape(Cin, RPAD * WROW)
    lo = (xpw - hi.reshape(Cin, RPAD, WROW).astype(jnp.float32))
    lo = lo.astype(jnp.bfloat16).reshape(Cin, RPAD * WROW)
    ew = expert_W.reshape(E * Cout, Cin, 3, 3)
    allw = jnp.concatenate([ew, shared_W], 0)
    wflat = jnp.transpose(allw, (0, 2, 3, 1)).reshape(NE, 9 * Cin)
    g = jnp.transpose(gate_W, (0, 2, 3, 1)).reshape(E, 9 * Cin)
    g_hi = g.astype(jnp.bfloat16)
    g_lo = (g - g_hi.astype(jnp.float32)).astype(jnp.bfloat16)
    wall = jnp.concatenate(
        [wflat.astype(jnp.bfloat16), g_hi, g_lo], 0)
    ebt = expert_b.T
    sb2 = shared_b[:, None]
    nt = 28
    out_flat = pl.pallas_call(
        _moe_body,
        grid=(nt,),
        in_specs=[
            pl.BlockSpec((Cin, RPAD * WROW), lambda i: (0, 0)),
            pl.BlockSpec((Cin, RPAD * WROW), lambda i: (0, 0)),
            pl.BlockSpec((NE + 16, 9 * Cin), lambda i: (0, 0)),
            pl.BlockSpec((Cout, E), lambda i: (0, 0)),
            pl.BlockSpec((Cout, 1), lambda i: (0, 0)),
        ],
        out_specs=pl.BlockSpec((Cout, M), lambda i: (0, i + 2)),
        out_shape=jax.ShapeDtypeStruct((Cout, (nt + 2) * M), jnp.float32),
    )(hi, lo, wall, ebt, sb2)
    out = out_flat.reshape(Cout, (nt + 2) * 8, WROW)[:, 16:16 + H, 1:1 + W]
    return out[None]


# speedup vs baseline: 1.1088x; 1.0462x over previous
import jax
import jax.numpy as jnp
from jax.experimental import pallas as pl

M = 4096
WROW = 256
NE = 864
RPAD = 256


def _moe_body(xh_ref, xl_ref, w_ref, ebt_ref, sb_ref, o_ref):
    i = pl.program_id(0)
    base = pl.multiple_of(i * M + 3712, 128)
    xwh = xh_ref[:, pl.ds(base, M + 768)]
    xwl = xl_ref[:, pl.ds(base, M + 768)]
    varh = [jax.lax.slice(xwh, (0, 127 + dx), (96, 127 + dx + M + 512))
            for dx in range(3)]
    varl = [jax.lax.slice(xwl, (0, 127 + dx), (96, 127 + dx + M + 512))
            for dx in range(3)]
    his, los = [], []
    for dy in range(3):
        for dx in range(3):
            his.append(jax.lax.slice(varh[dx], (0, dy * 256),
                                     (96, dy * 256 + M)))
            los.append(jax.lax.slice(varl[dx], (0, dy * 256),
                                     (96, dy * 256 + M)))
    x9 = jnp.concatenate(his, 0)
    x9l = jnp.concatenate(los, 0)
    acc = jax.lax.dot_general(w_ref[...], x9, (((1,), (0,)), ((), ())),
                              preferred_element_type=jnp.float32)
    zl = jax.lax.dot_general(w_ref[NE:NE + 16], x9l, (((1,), (0,)), ((), ())),
                             preferred_element_type=jnp.float32)
    z = acc[NE:NE + 8] + acc[NE + 8:NE + 16] + zl[0:8] + zl[8:16]
    neg = jnp.float32(-1e30)
    m1 = jnp.full((1, M), neg, jnp.float32)
    i1 = jnp.zeros((1, M), jnp.int32)
    for e in range(8):
        ze = z[e:e + 1]
        c = ze > m1
        m1 = jnp.where(c, ze, m1)
        i1 = jnp.where(c, e, i1)
    m2 = jnp.full((1, M), neg, jnp.float32)
    i2 = jnp.zeros((1, M), jnp.int32)
    for e in range(8):
        ze = z[e:e + 1]
        c = (ze > m2) & (i1 != e)
        m2 = jnp.where(c, ze, m2)
        i2 = jnp.where(c, e, i2)
    s1 = 1.0 / (1.0 + jnp.exp(-m1))
    s2 = 1.0 / (1.0 + jnp.exp(-m2))
    w1 = 1.0 / (1.0 + jnp.exp(s2 - s1))
    w2 = 1.0 - w1
    eio = jax.lax.broadcasted_iota(jnp.int32, (8, M), 0)
    sv = (jnp.where(eio == i1, w1, jnp.float32(0.0))
          + jnp.where(eio == i2, w2, jnp.float32(0.0)))
    out = acc[768:864]
    for e in range(8):
        out = out + acc[e * 96:(e + 1) * 96] * sv[e:e + 1]
    out = out + jax.lax.dot_general(ebt_ref[...], sv, (((1,), (0,)), ((), ())),
                                    preferred_element_type=jnp.float32)
    out = out + sb_ref[...]
    o_ref[...] = out


def kernel(x, gate_W, expert_W, expert_b, shared_W, shared_b):
    B, Cin, H, W = x.shape
    E, Cout = expert_W.shape[0], expert_W.shape[1]
    xpw = jnp.pad(x[0], ((0, 0), (16, 16), (1, WROW - W - 1)))
    hi = xpw.astype(jnp.bfloat16).reshape(Cin, RPAD * WROW)
    lo = (xpw - hi.reshape(Cin, RPAD, WROW).astype(jnp.float32))
    lo = lo.astype(jnp.bfloat16).reshape(Cin, RPAD * WROW)
    ew = expert_W.reshape(E * Cout, Cin, 3, 3)
    allw = jnp.concatenate([ew, shared_W], 0)
    wflat = jnp.transpose(allw, (0, 2, 3, 1)).reshape(NE, 9 * Cin)
    g = jnp.transpose(gate_W, (0, 2, 3, 1)).reshape(E, 9 * Cin)
    g_hi = g.astype(jnp.bfloat16)
    g_lo = (g - g_hi.astype(jnp.float32)).astype(jnp.bfloat16)
    wall = jnp.concatenate(
        [wflat.astype(jnp.bfloat16), g_hi, g_lo], 0)
    ebt = expert_b.T
    sb2 = shared_b[:, None]
    nt = 14
    out_flat = pl.pallas_call(
        _moe_body,
        grid=(nt,),
        in_specs=[
            pl.BlockSpec((Cin, RPAD * WROW), lambda i: (0, 0)),
            pl.BlockSpec((Cin, RPAD * WROW), lambda i: (0, 0)),
            pl.BlockSpec((NE + 16, 9 * Cin), lambda i: (0, 0)),
            pl.BlockSpec((Cout, E), lambda i: (0, 0)),
            pl.BlockSpec((Cout, 1), lambda i: (0, 0)),
        ],
        out_specs=pl.BlockSpec((Cout, M), lambda i: (0, i + 1)),
        out_shape=jax.ShapeDtypeStruct((Cout, (nt + 1) * M), jnp.float32),
    )(hi, lo, wall, ebt, sb2)
    out = out_flat.reshape(Cout, (nt + 1) * 16, WROW)[:, 16:16 + H, 1:1 + W]
    return out[None]


# speedup vs baseline: 1.2547x; 1.1316x over previous
import jax
import jax.numpy as jnp
from jax.experimental import pallas as pl

M = 4096
WROW = 256
NE = 864
RPAD = 256


def _moe_body(xh_ref, xl_ref, w_ref, ebt_ref, sb_ref, o_ref):
    i = pl.program_id(0)
    base = pl.multiple_of(i * M + 3712, 128)
    xwh = xh_ref[:, pl.ds(base, M + 768)]
    xwl = xl_ref[:, pl.ds(base, M + 768)]
    varh = [jax.lax.slice(xwh, (0, 127 + dx), (96, 127 + dx + M + 512))
            for dx in range(3)]
    varl = [jax.lax.slice(xwl, (0, 127 + dx), (96, 127 + dx + M + 512))
            for dx in range(3)]
    his, los = [], []
    for dy in range(3):
        for dx in range(3):
            his.append(jax.lax.slice(varh[dx], (0, dy * 256),
                                     (96, dy * 256 + M)))
            los.append(jax.lax.slice(varl[dx], (0, dy * 256),
                                     (96, dy * 256 + M)))
    x9 = jnp.concatenate(his, 0)
    x9l = jnp.concatenate(los, 0)
    acc = jax.lax.dot_general(w_ref[...], x9, (((1,), (0,)), ((), ())),
                              preferred_element_type=jnp.float32)
    zl = jax.lax.dot_general(w_ref[NE:NE + 16], x9l, (((1,), (0,)), ((), ())),
                             preferred_element_type=jnp.float32)
    z = acc[NE:NE + 8] + acc[NE + 8:NE + 16] + zl[0:8] + zl[8:16]
    neg = jnp.float32(-1e30)
    m1 = jnp.full((1, M), neg, jnp.float32)
    i1 = jnp.zeros((1, M), jnp.int32)
    for e in range(8):
        ze = z[e:e + 1]
        c = ze > m1
        m1 = jnp.where(c, ze, m1)
        i1 = jnp.where(c, e, i1)
    m2 = jnp.full((1, M), neg, jnp.float32)
    i2 = jnp.zeros((1, M), jnp.int32)
    for e in range(8):
        ze = z[e:e + 1]
        c = (ze > m2) & (i1 != e)
        m2 = jnp.where(c, ze, m2)
        i2 = jnp.where(c, e, i2)
    s1 = 1.0 / (1.0 + jnp.exp(-m1))
    s2 = 1.0 / (1.0 + jnp.exp(-m2))
    w1 = 1.0 / (1.0 + jnp.exp(s2 - s1))
    w2 = 1.0 - w1
    eio = jax.lax.broadcasted_iota(jnp.int32, (8, M), 0)
    sv = (jnp.where(eio == i1, w1, jnp.float32(0.0))
          + jnp.where(eio == i2, w2, jnp.float32(0.0)))
    out = acc[768:864]
    for e in range(8):
        out = out + acc[e * 96:(e + 1) * 96] * sv[e:e + 1]
    out = out + jax.lax.dot_general(ebt_ref[...], sv, (((1,), (0,)), ((), ())),
                                    preferred_element_type=jnp.float32)
    out = out + sb_ref[...]
    o3 = out.reshape(96, 16, 256)
    o_ref[...] = jax.lax.slice(o3, (0, 0, 1), (96, 16, 225))


def kernel(x, gate_W, expert_W, expert_b, shared_W, shared_b):
    B, Cin, H, W = x.shape
    E, Cout = expert_W.shape[0], expert_W.shape[1]
    xpw = jnp.pad(x[0], ((0, 0), (16, 16), (1, WROW - W - 1)))
    hi = xpw.astype(jnp.bfloat16).reshape(Cin, RPAD * WROW)
    lo = (xpw - hi.reshape(Cin, RPAD, WROW).astype(jnp.float32))
    lo = lo.astype(jnp.bfloat16).reshape(Cin, RPAD * WROW)
    ew = expert_W.reshape(E * Cout, Cin, 3, 3)
    allw = jnp.concatenate([ew, shared_W], 0)
    wflat = jnp.transpose(allw, (0, 2, 3, 1)).reshape(NE, 9 * Cin)
    g = jnp.transpose(gate_W, (0, 2, 3, 1)).reshape(E, 9 * Cin)
    g_hi = g.astype(jnp.bfloat16)
    g_lo = (g - g_hi.astype(jnp.float32)).astype(jnp.bfloat16)
    wall = jnp.concatenate(
        [wflat.astype(jnp.bfloat16), g_hi, g_lo], 0)
    ebt = expert_b.T
    sb2 = shared_b[:, None]
    nt = 14
    out_flat = pl.pallas_call(
        _moe_body,
        grid=(nt,),
        in_specs=[
            pl.BlockSpec((Cin, RPAD * WROW), lambda i: (0, 0)),
            pl.BlockSpec((Cin, RPAD * WROW), lambda i: (0, 0)),
            pl.BlockSpec((NE + 16, 9 * Cin), lambda i: (0, 0)),
            pl.BlockSpec((Cout, E), lambda i: (0, 0)),
            pl.BlockSpec((Cout, 1), lambda i: (0, 0)),
        ],
        out_specs=pl.BlockSpec((Cout, 16, W), lambda i: (0, i, 0)),
        out_shape=jax.ShapeDtypeStruct((Cout, H, W), jnp.float32),
    )(hi, lo, wall, ebt, sb2)
    return out_flat[None]
